# initial kernel scaffold (unmeasured)
import functools

import jax
import jax.numpy as jnp
from jax import lax
from jax.experimental import pallas as pl
from jax.experimental.pallas import tpu as pltpu

N_DEV = 8


def kernel(A, B):
    m, _ = A.shape
    _, n = B.shape
    ch = m // N_DEV

    def body(a_ref, b_ref, out_ref, rs_ref, rs_send, rs_recv, ag_send, ag_recv):
        my = lax.axis_index("i")
        left = lax.rem(my - 1 + N_DEV, N_DEV)
        right = lax.rem(my + 1, N_DEV)

        barrier = pltpu.get_barrier_semaphore()
        for nbr in (left, right):
            pl.semaphore_signal(
                barrier, inc=1, device_id=(nbr,),
                device_id_type=pl.DeviceIdType.MESH,
            )
        pl.semaphore_wait(barrier, 2)

        out_ref[...] = jnp.dot(
            a_ref[...], b_ref[...], preferred_element_type=jnp.float32
        )

        for s in range(N_DEV - 1):
            send_c = lax.rem(my - s + N_DEV, N_DEV)
            recv_c = lax.rem(my - s - 1 + N_DEV, N_DEV)
            rdma = pltpu.make_async_remote_copy(
                src_ref=out_ref.at[pl.ds(send_c * ch, ch), :],
                dst_ref=rs_ref.at[s],
                send_sem=rs_send.at[s],
                recv_sem=rs_recv.at[s],
                device_id=(right,),
                device_id_type=pl.DeviceIdType.MESH,
            )
            rdma.start()
            rdma.wait()
            out_ref[pl.ds(recv_c * ch, ch), :] = (
                out_ref[pl.ds(recv_c * ch, ch), :] + rs_ref[s]
            )

        for s in range(N_DEV - 1):
            send_c = lax.rem(my + 1 - s + N_DEV, N_DEV)
            rdma = pltpu.make_async_remote_copy(
                src_ref=out_ref.at[pl.ds(send_c * ch, ch), :],
                dst_ref=out_ref.at[pl.ds(send_c * ch, ch), :],
                send_sem=ag_send.at[s],
                recv_sem=ag_recv.at[s],
                device_id=(right,),
                device_id_type=pl.DeviceIdType.MESH,
            )
            rdma.start()
            rdma.wait()

        @functools.partial(
            pl.run_scoped, second_barrier=pltpu.SemaphoreType.REGULAR
        )
        def _(second_barrier):
            for nbr in (left, right):
                pl.semaphore_signal(
                    second_barrier, inc=1, device_id=(nbr,),
                    device_id_type=pl.DeviceIdType.MESH,
                )
            pl.semaphore_wait(second_barrier, 2)

    return pl.pallas_call(
        body,
        out_shape=jax.ShapeDtypeStruct((m, n), jnp.float32),
        in_specs=[
            pl.BlockSpec(memory_space=pltpu.VMEM),
            pl.BlockSpec(memory_space=pltpu.VMEM),
        ],
        out_specs=pl.BlockSpec(memory_space=pltpu.VMEM),
        scratch_shapes=[
            pltpu.VMEM((N_DEV - 1, ch, n), jnp.float32),
            pltpu.SemaphoreType.DMA((N_DEV - 1,)),
            pltpu.SemaphoreType.DMA((N_DEV - 1,)),
            pltpu.SemaphoreType.DMA((N_DEV - 1,)),
            pltpu.SemaphoreType.DMA((N_DEV - 1,)),
        ],
        compiler_params=pltpu.CompilerParams(collective_id=0),
    )(A, B)


# baseline (device time: 379609 ns/iter reference)
import functools

import jax
import jax.numpy as jnp
from jax import lax
from jax.experimental import pallas as pl
from jax.experimental.pallas import tpu as pltpu

N_DEV = 8


def kernel(A, B):
    m, _ = A.shape
    _, n = B.shape
    ch = m // N_DEV

    def body(a_ref, b_ref, out_ref, rs_ref, rs_send, rs_recv, ag_send, ag_recv):
        my = lax.axis_index("i")
        left = lax.rem(my - 1 + N_DEV, N_DEV)
        right = lax.rem(my + 1, N_DEV)

        barrier = pltpu.get_barrier_semaphore()
        for nbr in (left, right):
            pl.semaphore_signal(
                barrier, inc=1, device_id=(nbr,),
                device_id_type=pl.DeviceIdType.MESH,
            )
        pl.semaphore_wait(barrier, 2)

        out_ref[...] = jnp.dot(
            a_ref[...], b_ref[...], preferred_element_type=jnp.float32
        )

        for s in range(N_DEV - 1):
            send_c = lax.rem(my - s + N_DEV, N_DEV)
            recv_c = lax.rem(my - s - 1 + N_DEV, N_DEV)
            rdma = pltpu.make_async_remote_copy(
                src_ref=out_ref.at[pl.ds(send_c * ch, ch), :],
                dst_ref=rs_ref.at[s],
                send_sem=rs_send.at[s],
                recv_sem=rs_recv.at[s],
                device_id=(right,),
                device_id_type=pl.DeviceIdType.MESH,
            )
            rdma.start()
            rdma.wait()
            out_ref[pl.ds(recv_c * ch, ch), :] = (
                out_ref[pl.ds(recv_c * ch, ch), :] + rs_ref[s]
            )

        for s in range(N_DEV - 1):
            send_c = lax.rem(my + 1 - s + N_DEV, N_DEV)
            rdma = pltpu.make_async_remote_copy(
                src_ref=out_ref.at[pl.ds(send_c * ch, ch), :],
                dst_ref=out_ref.at[pl.ds(send_c * ch, ch), :],
                send_sem=ag_send.at[s],
                recv_sem=ag_recv.at[s],
                device_id=(right,),
                device_id_type=pl.DeviceIdType.MESH,
            )
            rdma.start()
            rdma.wait()

        @functools.partial(
            pl.run_scoped, second_barrier=pltpu.SemaphoreType.REGULAR
        )
        def _(second_barrier):
            for nbr in (left, right):
                pl.semaphore_signal(
                    second_barrier, inc=1, device_id=(nbr,),
                    device_id_type=pl.DeviceIdType.MESH,
                )
            pl.semaphore_wait(second_barrier, 2)

    return pl.pallas_call(
        body,
        out_shape=jax.ShapeDtypeStruct((m, n), jnp.float32),
        in_specs=[
            pl.BlockSpec(memory_space=pltpu.VMEM),
            pl.BlockSpec(memory_space=pltpu.VMEM),
        ],
        out_specs=pl.BlockSpec(memory_space=pltpu.VMEM),
        scratch_shapes=[
            pltpu.VMEM((N_DEV - 1, ch, n), jnp.float32),
            pltpu.SemaphoreType.DMA((N_DEV - 1,)),
            pltpu.SemaphoreType.DMA((N_DEV - 1,)),
            pltpu.SemaphoreType.DMA((N_DEV - 1,)),
            pltpu.SemaphoreType.DMA((N_DEV - 1,)),
        ],
        compiler_params=pltpu.CompilerParams(
            collective_id=0, vmem_limit_bytes=100 * 1024 * 1024
        ),
    )(A, B)


# device time: 159032 ns/iter; 2.3870x vs baseline; 2.3870x over previous
import functools

import jax
import jax.numpy as jnp
from jax import lax
from jax.experimental import pallas as pl
from jax.experimental.pallas import tpu as pltpu

N_DEV = 8
MASKS = (1, 3, 4)
PARTS = (704, 704, 640)
N_BF = 3


def kernel(A, B):
    m, _ = A.shape
    _, n = B.shape
    assert sum(PARTS) == m
    base = (0, PARTS[0], PARTS[0] + PARTS[1])
    perm = tuple(tuple((b + s) % N_BF for s in range(3)) for b in range(N_BF))

    def body(a_ref, b_ref, out_ref, *scratch):
        bufs = [list(scratch[3 * b : 3 * b + 3]) for b in range(N_BF)]
        rs_send, rs_recv, ag_send, ag_recv = scratch[9:]

        my = lax.axis_index("i")
        bit_y = lax.shift_right_logical(my, 1) & 1
        bit_z = lax.shift_right_logical(my, 2) & 1
        bit_x = bit_y ^ (my & 1)
        bits = (bit_x, bit_y, bit_z)
        left = lax.rem(my - 1 + N_DEV, N_DEV)
        right = lax.rem(my + 1, N_DEV)

        barrier = pltpu.get_barrier_semaphore()
        for nbr in (left, right):
            pl.semaphore_signal(
                barrier, inc=1, device_id=(nbr,),
                device_id_type=pl.DeviceIdType.MESH,
            )
        pl.semaphore_wait(barrier, 2)

        out_ref[...] = jnp.dot(
            a_ref[...], b_ref[...], preferred_element_type=jnp.float32
        )

        start = [jnp.int32(base[b]) for b in range(N_BF)]
        size = [PARTS[b] for b in range(N_BF)]
        for s in range(3):
            rdmas = []
            for b in range(N_BF):
                ax = perm[b][s]
                half = size[b] // 2
                mb = bits[ax]
                keep = start[b] + mb * half
                send = start[b] + (1 - mb) * half
                partner = my ^ MASKS[ax]
                rdma = pltpu.make_async_remote_copy(
                    src_ref=out_ref.at[pl.ds(send, half), :],
                    dst_ref=bufs[b][s],
                    send_sem=rs_send.at[b, s],
                    recv_sem=rs_recv.at[b, s],
                    device_id=(partner,),
                    device_id_type=pl.DeviceIdType.MESH,
                )
                rdma.start()
                rdmas.append((rdma, keep, half))
                start[b] = keep
                size[b] = half
            for b, (rdma, keep, half) in enumerate(rdmas):
                rdma.wait()
                out_ref[pl.ds(keep, half), :] = (
                    out_ref[pl.ds(keep, half), :] + bufs[b][s][...]
                )

        for t in range(3):
            rdmas = []
            for b in range(N_BF):
                ax = perm[b][2 - t]
                partner = my ^ MASKS[ax]
                rdma = pltpu.make_async_remote_copy(
                    src_ref=out_ref.at[pl.ds(start[b], size[b]), :],
                    dst_ref=out_ref.at[pl.ds(start[b], size[b]), :],
                    send_sem=ag_send.at[b, t],
                    recv_sem=ag_recv.at[b, t],
                    device_id=(partner,),
                    device_id_type=pl.DeviceIdType.MESH,
                )
                rdma.start()
                rdmas.append(rdma)
                start[b] = start[b] - bits[ax] * size[b]
                size[b] = 2 * size[b]
            for rdma in rdmas:
                rdma.wait()

        @functools.partial(
            pl.run_scoped, second_barrier=pltpu.SemaphoreType.REGULAR
        )
        def _(second_barrier):
            for nbr in (left, right):
                pl.semaphore_signal(
                    second_barrier, inc=1, device_id=(nbr,),
                    device_id_type=pl.DeviceIdType.MESH,
                )
            pl.semaphore_wait(second_barrier, 2)

    rs_bufs = [
        pltpu.VMEM((PARTS[b] // (2 ** (s + 1)), n), jnp.float32)
        for b in range(N_BF)
        for s in range(3)
    ]
    return pl.pallas_call(
        body,
        out_shape=jax.ShapeDtypeStruct((m, n), jnp.float32),
        in_specs=[
            pl.BlockSpec(memory_space=pltpu.VMEM),
            pl.BlockSpec(memory_space=pltpu.VMEM),
        ],
        out_specs=pl.BlockSpec(memory_space=pltpu.VMEM),
        scratch_shapes=rs_bufs
        + [
            pltpu.SemaphoreType.DMA((N_BF, 3)),
            pltpu.SemaphoreType.DMA((N_BF, 3)),
            pltpu.SemaphoreType.DMA((N_BF, 3)),
            pltpu.SemaphoreType.DMA((N_BF, 3)),
        ],
        compiler_params=pltpu.CompilerParams(
            collective_id=0, vmem_limit_bytes=100 * 1024 * 1024
        ),
    )(A, B)


# device time: 152246 ns/iter; 2.4934x vs baseline; 1.0446x over previous
import functools

import jax
import jax.numpy as jnp
from jax import lax
from jax.experimental import pallas as pl
from jax.experimental.pallas import tpu as pltpu

N_DEV = 8
MASKS = (1, 3, 4)
PARTS = (704, 704, 640)
N_BF = 3


def kernel(A, B):
    m, _ = A.shape
    _, n = B.shape
    assert sum(PARTS) == m
    base = (0, PARTS[0], PARTS[0] + PARTS[1])
    perm = tuple(tuple((b + s) % N_BF for s in range(3)) for b in range(N_BF))

    def body(a_ref, b_ref, out_ref, *scratch):
        bufs = [list(scratch[3 * b : 3 * b + 3]) for b in range(N_BF)]
        rs_send, rs_recv, ag_send, ag_recv = scratch[9:]

        my = lax.axis_index("i")
        bit_y = lax.shift_right_logical(my, 1) & 1
        bit_z = lax.shift_right_logical(my, 2) & 1
        bit_x = bit_y ^ (my & 1)
        bits = (bit_x, bit_y, bit_z)
        left = lax.rem(my - 1 + N_DEV, N_DEV)
        right = lax.rem(my + 1, N_DEV)

        barrier = pltpu.get_barrier_semaphore()
        for nbr in (left, right):
            pl.semaphore_signal(
                barrier, inc=1, device_id=(nbr,),
                device_id_type=pl.DeviceIdType.MESH,
            )
        pl.semaphore_wait(barrier, 2)

        def partial_rows(r0, nrows):
            out_ref[pl.ds(r0, nrows), :] = jnp.dot(
                a_ref[pl.ds(r0, nrows), :],
                b_ref[...],
                preferred_element_type=jnp.float32,
            )

        def mk_rs(b, s, send, half, partner):
            return pltpu.make_async_remote_copy(
                src_ref=out_ref.at[pl.ds(send, half), :],
                dst_ref=bufs[b][s],
                send_sem=rs_send.at[b, s],
                recv_sem=rs_recv.at[b, s],
                device_id=(partner,),
                device_id_type=pl.DeviceIdType.MESH,
            )

        def mk_ag(b, t, r0, nrows, partner):
            return pltpu.make_async_remote_copy(
                src_ref=out_ref.at[pl.ds(r0, nrows), :],
                dst_ref=out_ref.at[pl.ds(r0, nrows), :],
                send_sem=ag_send.at[b, t],
                recv_sem=ag_recv.at[b, t],
                device_id=(partner,),
                device_id_type=pl.DeviceIdType.MESH,
            )

        start = [jnp.int32(base[b]) for b in range(N_BF)]
        size = [PARTS[b] for b in range(N_BF)]

        rdmas = []
        keeps = []
        for b in range(N_BF):
            ax = perm[b][0]
            half = size[b] // 2
            mb = bits[ax]
            keep = start[b] + mb * half
            send = start[b] + (1 - mb) * half
            partial_rows(send, half)
            rdma = mk_rs(b, 0, send, half, my ^ MASKS[ax])
            rdma.start()
            rdmas.append((rdma, keep, half))
            keeps.append((keep, half))
            start[b] = keep
            size[b] = half
        for keep, half in keeps:
            partial_rows(keep, half)

        for s in range(3):
            nxt = []
            for b in range(N_BF):
                rdma, keep, half = rdmas[b]
                rdma.wait()
                out_ref[pl.ds(keep, half), :] = (
                    out_ref[pl.ds(keep, half), :] + bufs[b][s][...]
                )
                if s < 2:
                    ax = perm[b][s + 1]
                    h2 = size[b] // 2
                    mb = bits[ax]
                    k2 = start[b] + mb * h2
                    snd = start[b] + (1 - mb) * h2
                    r2 = mk_rs(b, s + 1, snd, h2, my ^ MASKS[ax])
                    r2.start()
                    nxt.append((r2, k2, h2))
                    start[b] = k2
                    size[b] = h2
                else:
                    ax = perm[b][2]
                    r2 = mk_ag(b, 0, start[b], size[b], my ^ MASKS[ax])
                    r2.start()
                    nxt.append((r2, ax))
            rdmas = nxt

        for t in range(3):
            nxt = []
            for b in range(N_BF):
                rdma, ax = rdmas[b]
                rdma.wait()
                start[b] = start[b] - bits[ax] * size[b]
                size[b] = 2 * size[b]
                if t < 2:
                    ax2 = perm[b][1 - t]
                    r2 = mk_ag(b, t + 1, start[b], size[b], my ^ MASKS[ax2])
                    r2.start()
                    nxt.append((r2, ax2))
            rdmas = nxt

        @functools.partial(
            pl.run_scoped, second_barrier=pltpu.SemaphoreType.REGULAR
        )
        def _(second_barrier):
            for nbr in (left, right):
                pl.semaphore_signal(
                    second_barrier, inc=1, device_id=(nbr,),
                    device_id_type=pl.DeviceIdType.MESH,
                )
            pl.semaphore_wait(second_barrier, 2)

    rs_bufs = [
        pltpu.VMEM((PARTS[b] // (2 ** (s + 1)), n), jnp.float32)
        for b in range(N_BF)
        for s in range(3)
    ]
    return pl.pallas_call(
        body,
        out_shape=jax.ShapeDtypeStruct((m, n), jnp.float32),
        in_specs=[
            pl.BlockSpec(memory_space=pltpu.VMEM),
            pl.BlockSpec(memory_space=pltpu.VMEM),
        ],
        out_specs=pl.BlockSpec(memory_space=pltpu.VMEM),
        scratch_shapes=rs_bufs
        + [
            pltpu.SemaphoreType.DMA((N_BF, 3)),
            pltpu.SemaphoreType.DMA((N_BF, 3)),
            pltpu.SemaphoreType.DMA((N_BF, 3)),
            pltpu.SemaphoreType.DMA((N_BF, 3)),
        ],
        compiler_params=pltpu.CompilerParams(
            collective_id=0, vmem_limit_bytes=100 * 1024 * 1024
        ),
    )(A, B)


# device time: 98886 ns/iter; 3.8389x vs baseline; 1.5396x over previous
import functools

import jax
import jax.numpy as jnp
from jax import lax
from jax.experimental import pallas as pl
from jax.experimental.pallas import tpu as pltpu

N_DEV = 8
MASKS = (1, 3, 4)
PARTS = (704, 704, 640)
N_BF = 3


def kernel(A, B):
    m, _ = A.shape
    _, n = B.shape
    assert sum(PARTS) == m
    base = (0, PARTS[0], PARTS[0] + PARTS[1])
    perm = tuple(tuple((b + s) % N_BF for s in range(3)) for b in range(N_BF))

    def body(a_ref, b_ref, out_ref, *scratch):
        rs_rx = [list(scratch[3 * b : 3 * b + 3]) for b in range(N_BF)]
        rs_tx = [list(scratch[9 + 3 * b : 12 + 3 * b]) for b in range(N_BF)]
        ag_rx = [list(scratch[18 + 3 * b : 21 + 3 * b]) for b in range(N_BF)]
        ag_tx = [list(scratch[27 + 3 * b : 30 + 3 * b]) for b in range(N_BF)]
        rs_send, rs_recv, ag_send, ag_recv = scratch[36:]

        my = lax.axis_index("i")
        bit_y = lax.shift_right_logical(my, 1) & 1
        bit_z = lax.shift_right_logical(my, 2) & 1
        bit_x = bit_y ^ (my & 1)
        bits = (bit_x, bit_y, bit_z)
        left = lax.rem(my - 1 + N_DEV, N_DEV)
        right = lax.rem(my + 1, N_DEV)

        barrier = pltpu.get_barrier_semaphore()
        for nbr in (left, right):
            pl.semaphore_signal(
                barrier, inc=1, device_id=(nbr,),
                device_id_type=pl.DeviceIdType.MESH,
            )
        pl.semaphore_wait(barrier, 2)

        def mk_rs(b, s, partner):
            return pltpu.make_async_remote_copy(
                src_ref=rs_tx[b][s],
                dst_ref=rs_rx[b][s],
                send_sem=rs_send.at[b, s],
                recv_sem=rs_recv.at[b, s],
                device_id=(partner,),
                device_id_type=pl.DeviceIdType.MESH,
            )

        def mk_ag(b, t, partner):
            return pltpu.make_async_remote_copy(
                src_ref=ag_tx[b][t],
                dst_ref=ag_rx[b][t],
                send_sem=ag_send.at[b, t],
                recv_sem=ag_recv.at[b, t],
                device_id=(partner,),
                device_id_type=pl.DeviceIdType.MESH,
            )

        start = [jnp.int32(base[b]) for b in range(N_BF)]
        size = [PARTS[b] for b in range(N_BF)]

        rdmas = []
        keeps = []
        for b in range(N_BF):
            ax = perm[b][0]
            half = size[b] // 2
            mb = bits[ax]
            keep = start[b] + mb * half
            send = start[b] + (1 - mb) * half
            rs_tx[b][0][...] = jnp.dot(
                a_ref[pl.ds(send, half), :],
                b_ref[...],
                preferred_element_type=jnp.float32,
            ).astype(jnp.bfloat16)
            rdma = mk_rs(b, 0, my ^ MASKS[ax])
            rdma.start()
            rdmas.append((rdma, keep, half))
            keeps.append((keep, half))
            start[b] = keep
            size[b] = half
        for keep, half in keeps:
            out_ref[pl.ds(keep, half), :] = jnp.dot(
                a_ref[pl.ds(keep, half), :],
                b_ref[...],
                preferred_element_type=jnp.float32,
            )

        for s in range(3):
            nxt = []
            for b in range(N_BF):
                rdma, keep, half = rdmas[b]
                rdma.wait()
                out_ref[pl.ds(keep, half), :] = (
                    out_ref[pl.ds(keep, half), :]
                    + rs_rx[b][s][...].astype(jnp.float32)
                )
                if s < 2:
                    ax = perm[b][s + 1]
                    h2 = size[b] // 2
                    mb = bits[ax]
                    k2 = start[b] + mb * h2
                    snd = start[b] + (1 - mb) * h2
                    rs_tx[b][s + 1][...] = out_ref[
                        pl.ds(snd, h2), :
                    ].astype(jnp.bfloat16)
                    r2 = mk_rs(b, s + 1, my ^ MASKS[ax])
                    r2.start()
                    nxt.append((r2, k2, h2))
                    start[b] = k2
                    size[b] = h2
                else:
                    ax = perm[b][2]
                    ag_tx[b][0][...] = out_ref[
                        pl.ds(start[b], size[b]), :
                    ].astype(jnp.bfloat16)
                    r2 = mk_ag(b, 0, my ^ MASKS[ax])
                    r2.start()
                    nxt.append((r2, ax))
            rdmas = nxt

        for t in range(3):
            nxt = []
            for b in range(N_BF):
                rdma, ax = rdmas[b]
                rdma.wait()
                sib = start[b] - bits[ax] * size[b] + (1 - bits[ax]) * size[b]
                out_ref[pl.ds(sib, size[b]), :] = ag_rx[b][t][...].astype(
                    jnp.float32
                )
                start[b] = start[b] - bits[ax] * size[b]
                size[b] = 2 * size[b]
                if t < 2:
                    ax2 = perm[b][1 - t]
                    ag_tx[b][t + 1][...] = out_ref[
                        pl.ds(start[b], size[b]), :
                    ].astype(jnp.bfloat16)
                    r2 = mk_ag(b, t + 1, my ^ MASKS[ax2])
                    r2.start()
                    nxt.append((r2, ax2))
            rdmas = nxt

        @functools.partial(
            pl.run_scoped, second_barrier=pltpu.SemaphoreType.REGULAR
        )
        def _(second_barrier):
            for nbr in (left, right):
                pl.semaphore_signal(
                    second_barrier, inc=1, device_id=(nbr,),
                    device_id_type=pl.DeviceIdType.MESH,
                )
            pl.semaphore_wait(second_barrier, 2)

    rs_shapes = [
        pltpu.VMEM((PARTS[b] // (2 ** (s + 1)), n), jnp.bfloat16)
        for b in range(N_BF)
        for s in range(3)
    ]
    ag_shapes = [
        pltpu.VMEM((PARTS[b] // (2 ** (3 - t)), n), jnp.bfloat16)
        for b in range(N_BF)
        for t in range(3)
    ]
    return pl.pallas_call(
        body,
        out_shape=jax.ShapeDtypeStruct((m, n), jnp.float32),
        in_specs=[
            pl.BlockSpec(memory_space=pltpu.VMEM),
            pl.BlockSpec(memory_space=pltpu.VMEM),
        ],
        out_specs=pl.BlockSpec(memory_space=pltpu.VMEM),
        scratch_shapes=rs_shapes
        + rs_shapes
        + ag_shapes
        + ag_shapes
        + [
            pltpu.SemaphoreType.DMA((N_BF, 3)),
            pltpu.SemaphoreType.DMA((N_BF, 3)),
            pltpu.SemaphoreType.DMA((N_BF, 3)),
            pltpu.SemaphoreType.DMA((N_BF, 3)),
        ],
        compiler_params=pltpu.CompilerParams(
            collective_id=0, vmem_limit_bytes=100 * 1024 * 1024
        ),
    )(A, B)
